# baseline (speedup 1.0000x reference)
import functools

import jax
import jax.numpy as jnp
from jax import lax
from jax.experimental import pallas as pl
from jax.experimental.pallas import tpu as pltpu
from jax.experimental.pallas import tpu_sc as plsc

_N_FINE = 200000
_N_COARSE = 50000
_K = 9
_F = 128

_info = plsc.get_sparse_core_info()
_NC = _info.num_cores
_NS = _info.num_subcores
_NW = _NC * _NS

_P = 1
_SLICE_C = _N_COARSE // _P
_HALF_C = _SLICE_C // 2
_NQ = _K * _HALF_C

_CHQ = 120
_N_CHUNKS = _NQ // _CHQ
_MAXC = -(-_N_CHUNKS // _NW)

_mesh = plsc.VectorSubcoreMesh(core_axis_name="c", subcore_axis_name="s")


@functools.partial(
    pl.kernel,
    mesh=_mesh,
    out_type=jax.ShapeDtypeStruct((_NQ, _F), jnp.int32),
    scratch_types=[
        pltpu.VMEM((_MAXC * _CHQ,), jnp.int32),
        pltpu.VMEM((_MAXC * _CHQ,), jnp.int32),
        pltpu.VMEM((_CHQ, _F), jnp.int32),
        pltpu.VMEM((_CHQ, _F), jnp.int32),
        pltpu.VMEM((_CHQ, _F), jnp.int32),
        pltpu.VMEM((_CHQ, _F), jnp.int32),
        pltpu.VMEM((_CHQ, _F), jnp.int32),
        pltpu.VMEM((_CHQ, _F), jnp.int32),
        pltpu.SemaphoreType.DMA,
        pltpu.SemaphoreType.DMA,
        pltpu.SemaphoreType.DMA,
        pltpu.SemaphoreType.DMA,
    ],
)
def _sc_gather_pack(lv_hbm, idxlo_hbm, idxhi_hbm, out_hbm,
                    ilo_v, ihi_v, bl0, bh0, bl1, bh1, pk0, pk1,
                    gsem0, gsem1, wsem0, wsem1):
    wid = lax.axis_index("s") * _NC + lax.axis_index("c")
    c0 = wid * _N_CHUNKS // _NW
    n = (wid + 1) * _N_CHUNKS // _NW - c0
    base0 = c0 * _CHQ
    pltpu.sync_copy(idxlo_hbm.at[pl.ds(base0, _MAXC * _CHQ)], ilo_v)
    pltpu.sync_copy(idxhi_hbm.at[pl.ds(base0, _MAXC * _CHQ)], ihi_v)

    def _start_gather(i, bl, bh, gsem):
        pltpu.async_copy(
            lv_hbm.at[ilo_v.at[pl.ds(i * _CHQ, _CHQ)]], bl, gsem)
        pltpu.async_copy(
            lv_hbm.at[ihi_v.at[pl.ds(i * _CHQ, _CHQ)]], bh, gsem)

    def _wait_gather(bl, bh, gsem):
        pltpu.make_async_copy(lv_hbm.at[pl.ds(0, _CHQ)], bl, gsem).wait()
        pltpu.make_async_copy(lv_hbm.at[pl.ds(0, _CHQ)], bh, gsem).wait()

    def _start_write(i, pk, wsem):
        pltpu.async_copy(pk, out_hbm.at[pl.ds(base0 + i * _CHQ, _CHQ)], wsem)

    def _wait_write(pk, wsem):
        pltpu.make_async_copy(out_hbm.at[pl.ds(0, _CHQ)], pk, wsem).wait()

    def _pair(ai, bi):
        ar = ai + jnp.int32(0x8000)
        br = bi + jnp.int32(0x8000)
        return (lax.shift_right_logical(ar, 16)
                | (br & jnp.int32(-65536)))

    def _pack(bl, bh, pk):
        def row(q, carry):
            for g in range(8):
                pk[q, pl.ds(16 * g, 16)] = _pair(
                    bl[q, pl.ds(16 * g, 16)],
                    bh[q, pl.ds(16 * g, 16)])
            return carry
        lax.fori_loop(0, _CHQ, row, 0)

    _start_gather(0, bl0, bh0, gsem0)

    def body(i, carry):
        even = lax.rem(i, 2) == 0

        @pl.when(even)
        def _even():
            _wait_gather(bl0, bh0, gsem0)

            @pl.when(i + 1 < n)
            def _next():
                _start_gather(i + 1, bl1, bh1, gsem1)

            @pl.when(i >= 2)
            def _w():
                _wait_write(pk0, wsem0)
            _pack(bl0, bh0, pk0)
            _start_write(i, pk0, wsem0)

        @pl.when(jnp.logical_not(even))
        def _odd():
            _wait_gather(bl1, bh1, gsem1)

            @pl.when(i + 1 < n)
            def _next():
                _start_gather(i + 1, bl0, bh0, gsem0)

            @pl.when(i >= 2)
            def _w():
                _wait_write(pk1, wsem1)
            _pack(bl1, bh1, pk1)
            _start_write(i, pk1, wsem1)

        return carry

    lax.fori_loop(0, n, body, 0)
    _wait_write(pk0, wsem0)
    _wait_write(pk1, wsem1)


_BM = 1000


def _mm_body(g_ref, w_ref, b_ref, o_ref):
    k = pl.program_id(1)

    @pl.when(k == 0)
    def _init():
        o_ref[...] = jnp.broadcast_to(b_ref[...], (2, _BM, _F))

    x = g_ref[...]
    al = lax.bitcast_convert_type(x << 16, jnp.float32)
    ah = lax.bitcast_convert_type(x & jnp.int32(-65536), jnp.float32)
    o_ref[0] += jnp.dot(al, w_ref[0], preferred_element_type=jnp.float32)
    o_ref[1] += jnp.dot(ah, w_ref[0], preferred_element_type=jnp.float32)

    @pl.when(k == _K - 1)
    def _relu():
        o_ref[...] = jnp.maximum(o_ref[...], 0.0)


def _tc_matmul(g2, w3, b2d):
    out3 = pl.pallas_call(
        _mm_body,
        grid=(_HALF_C // _BM, _K),
        in_specs=[
            pl.BlockSpec((_BM, _F), lambda i, k: (k * (_HALF_C // _BM) + i, 0)),
            pl.BlockSpec((1, _F, _F), lambda i, k: (k, 0, 0)),
            pl.BlockSpec((1, _F), lambda i, k: (0, 0)),
        ],
        out_specs=pl.BlockSpec((2, _BM, _F), lambda i, k: (0, i, 0)),
        out_shape=jax.ShapeDtypeStruct((2, _HALF_C, _F), jnp.float32),
    )(g2, w3, b2d)
    return out3.reshape(_SLICE_C, _F)


def kernel(lv, ls_neighbors, W, b):
    lv_i = lax.bitcast_convert_type(lv, jnp.int32)
    idx_t = ls_neighbors.astype(jnp.int32).T
    w3 = W.reshape(_K, _F, _F)
    b2d = b.reshape(1, _F)
    gs = []
    for p in range(_P):
        sl = lax.slice_in_dim(idx_t, p * _SLICE_C, (p + 1) * _SLICE_C, axis=1)
        idx_lo = sl[:, :_HALF_C].reshape(_NQ)
        idx_hi = sl[:, _HALF_C:].reshape(_NQ)
        gs.append(_sc_gather_pack(lv_i, idx_lo, idx_hi))
    outs = [_tc_matmul(g2, w3, b2d) for g2 in gs]
    out = jnp.concatenate(outs, axis=0)
    return (out, ls_neighbors)


# speedup vs baseline: 1.1187x; 1.1187x over previous
import functools

import jax
import jax.numpy as jnp
from jax import lax
from jax.experimental import pallas as pl
from jax.experimental.pallas import tpu as pltpu
from jax.experimental.pallas import tpu_sc as plsc

_N_FINE = 200000
_N_COARSE = 50000
_K = 9
_F = 128

_info = plsc.get_sparse_core_info()
_NC = _info.num_cores
_NS = _info.num_subcores
_NW = _NC * _NS

_P = 5
_SLICE_C = _N_COARSE // _P
_HALF_C = _SLICE_C // 2
_NQ = _K * _HALF_C

_CHQ = 120
_N_CHUNKS = _NQ // _CHQ
_MAXC = -(-_N_CHUNKS // _NW)

_mesh = plsc.VectorSubcoreMesh(core_axis_name="c", subcore_axis_name="s")


@functools.partial(
    pl.kernel,
    mesh=_mesh,
    out_type=jax.ShapeDtypeStruct((_NQ, _F), jnp.int32),
    scratch_types=[
        pltpu.VMEM((_MAXC * _CHQ,), jnp.int32),
        pltpu.VMEM((_MAXC * _CHQ,), jnp.int32),
        pltpu.VMEM((_CHQ, _F), jnp.int32),
        pltpu.VMEM((_CHQ, _F), jnp.int32),
        pltpu.VMEM((_CHQ, _F), jnp.int32),
        pltpu.VMEM((_CHQ, _F), jnp.int32),
        pltpu.VMEM((_CHQ, _F), jnp.int32),
        pltpu.VMEM((_CHQ, _F), jnp.int32),
        pltpu.SemaphoreType.DMA,
        pltpu.SemaphoreType.DMA,
        pltpu.SemaphoreType.DMA,
        pltpu.SemaphoreType.DMA,
    ],
)
def _sc_gather_pack(lv_hbm, idxlo_hbm, idxhi_hbm, out_hbm,
                    ilo_v, ihi_v, bl0, bh0, bl1, bh1, pk0, pk1,
                    gsem0, gsem1, wsem0, wsem1):
    wid = lax.axis_index("s") * _NC + lax.axis_index("c")
    c0 = wid * _N_CHUNKS // _NW
    n = (wid + 1) * _N_CHUNKS // _NW - c0
    base0 = c0 * _CHQ
    pltpu.sync_copy(idxlo_hbm.at[pl.ds(base0, _MAXC * _CHQ)], ilo_v)
    pltpu.sync_copy(idxhi_hbm.at[pl.ds(base0, _MAXC * _CHQ)], ihi_v)

    def _start_gather(i, bl, bh, gsem):
        pltpu.async_copy(
            lv_hbm.at[ilo_v.at[pl.ds(i * _CHQ, _CHQ)]], bl, gsem)
        pltpu.async_copy(
            lv_hbm.at[ihi_v.at[pl.ds(i * _CHQ, _CHQ)]], bh, gsem)

    def _wait_gather(bl, bh, gsem):
        pltpu.make_async_copy(lv_hbm.at[pl.ds(0, _CHQ)], bl, gsem).wait()
        pltpu.make_async_copy(lv_hbm.at[pl.ds(0, _CHQ)], bh, gsem).wait()

    def _start_write(i, pk, wsem):
        pltpu.async_copy(pk, out_hbm.at[pl.ds(base0 + i * _CHQ, _CHQ)], wsem)

    def _wait_write(pk, wsem):
        pltpu.make_async_copy(out_hbm.at[pl.ds(0, _CHQ)], pk, wsem).wait()

    def _pair(ai, bi):
        ar = ai + jnp.int32(0x8000)
        br = bi + jnp.int32(0x8000)
        return (lax.shift_right_logical(ar, 16)
                | (br & jnp.int32(-65536)))

    def _pack(bl, bh, pk):
        def row(q, carry):
            for g in range(8):
                pk[q, pl.ds(16 * g, 16)] = _pair(
                    bl[q, pl.ds(16 * g, 16)],
                    bh[q, pl.ds(16 * g, 16)])
            return carry
        lax.fori_loop(0, _CHQ, row, 0)

    _start_gather(0, bl0, bh0, gsem0)

    def body(i, carry):
        even = lax.rem(i, 2) == 0

        @pl.when(even)
        def _even():
            _wait_gather(bl0, bh0, gsem0)

            @pl.when(i + 1 < n)
            def _next():
                _start_gather(i + 1, bl1, bh1, gsem1)

            @pl.when(i >= 2)
            def _w():
                _wait_write(pk0, wsem0)
            _pack(bl0, bh0, pk0)
            _start_write(i, pk0, wsem0)

        @pl.when(jnp.logical_not(even))
        def _odd():
            _wait_gather(bl1, bh1, gsem1)

            @pl.when(i + 1 < n)
            def _next():
                _start_gather(i + 1, bl0, bh0, gsem0)

            @pl.when(i >= 2)
            def _w():
                _wait_write(pk1, wsem1)
            _pack(bl1, bh1, pk1)
            _start_write(i, pk1, wsem1)

        return carry

    lax.fori_loop(0, n, body, 0)
    _wait_write(pk0, wsem0)
    _wait_write(pk1, wsem1)


_BM = 1000


def _mm_body(g_ref, w_ref, b_ref, o_ref):
    k = pl.program_id(1)

    @pl.when(k == 0)
    def _init():
        o_ref[...] = jnp.broadcast_to(b_ref[...], (2, _BM, _F))

    x = g_ref[...]
    al = lax.bitcast_convert_type(x << 16, jnp.float32).astype(jnp.bfloat16)
    ah = lax.bitcast_convert_type(
        x & jnp.int32(-65536), jnp.float32).astype(jnp.bfloat16)
    o_ref[0] += jnp.dot(al, w_ref[0], preferred_element_type=jnp.float32)
    o_ref[1] += jnp.dot(ah, w_ref[0], preferred_element_type=jnp.float32)

    @pl.when(k == _K - 1)
    def _relu():
        o_ref[...] = jnp.maximum(o_ref[...], 0.0)


def _tc_matmul(g2, w3, b2d):
    out3 = pl.pallas_call(
        _mm_body,
        grid=(_HALF_C // _BM, _K),
        in_specs=[
            pl.BlockSpec((_BM, _F), lambda i, k: (k * (_HALF_C // _BM) + i, 0)),
            pl.BlockSpec((1, _F, _F), lambda i, k: (k, 0, 0)),
            pl.BlockSpec((1, _F), lambda i, k: (0, 0)),
        ],
        out_specs=pl.BlockSpec((2, _BM, _F), lambda i, k: (0, i, 0)),
        out_shape=jax.ShapeDtypeStruct((2, _HALF_C, _F), jnp.float32),
    )(g2, w3, b2d)
    return out3.reshape(_SLICE_C, _F)


def kernel(lv, ls_neighbors, W, b):
    lv_i = lax.bitcast_convert_type(lv, jnp.int32)
    idx_t = ls_neighbors.astype(jnp.int32).T
    w3 = W.astype(jnp.bfloat16).reshape(_K, _F, _F)
    b2d = b.reshape(1, _F)
    gs = []
    for p in range(_P):
        sl = lax.slice_in_dim(idx_t, p * _SLICE_C, (p + 1) * _SLICE_C, axis=1)
        idx_lo = sl[:, :_HALF_C].reshape(_NQ)
        idx_hi = sl[:, _HALF_C:].reshape(_NQ)
        gs.append(_sc_gather_pack(lv_i, idx_lo, idx_hi))
    outs = [_tc_matmul(g2, w3, b2d) for g2 in gs]
    out = jnp.concatenate(outs, axis=0)
    return (out, ls_neighbors)
